# Initial kernel scaffold; baseline (speedup 1.0000x reference)
import jax
import jax.numpy as jnp
from jax.experimental import pallas as pl


def kernel(R_U, R_V, A_U, A_V, emb, uW1, ub1, uW2, ub2, uW3, ub3, iW1, ib1, iW2, ib2, iW3, ib3, ug1W, ug1as, ug1ad, ug1b, ug2W, ug2as, ug2ad, ug2b, ig1W, ig1as, ig1ad, ig1b, ig2W, ig2as, ig2ad, ig2b):
    raise NotImplementedError("write your pallas kernel here")


# speedup vs baseline: 57.9659x; 57.9659x over previous
import functools

import jax
import jax.numpy as jnp
from jax import lax
from jax.experimental import pallas as pl
from jax.experimental.pallas import tpu as pltpu
from jax.experimental.pallas import tpu_sc as plsc

N_USERS = 2048
N_ITEMS = 1024
D = 16
H1 = 256
H2 = 64
NR = 6
E_U = 65536
E_V = 32768


def _encoder_body(emb_ref, R_ref, W1r_ref, b1_ref, W2_ref, b2_ref, W3_ref,
                  b3_ref, out_ref, acc_ref):
    j = pl.program_id(0)
    nj = pl.num_programs(0)

    @pl.when(j == 0)
    def _():
        acc_ref[...] = jnp.zeros_like(acc_ref)

    w = W1r_ref[...]
    Rb = R_ref[...]
    acc = acc_ref[...]
    for r in range(NR):
        m_r = emb_ref[r, 0] * w[:, 0, :]
        for d in range(1, D):
            m_r = m_r + emb_ref[r, d] * w[:, d, :]
        mask = (Rb == r).astype(jnp.float32)
        acc = acc + jnp.dot(mask, m_r, preferred_element_type=jnp.float32)
    acc_ref[...] = acc

    @pl.when(j == nj - 1)
    def _():
        h = jnp.maximum(acc + b1_ref[...], 0.0)
        h = jnp.maximum(
            jnp.dot(h, W2_ref[...], preferred_element_type=jnp.float32)
            + b2_ref[...], 0.0)
        out_ref[...] = (
            jnp.dot(h, W3_ref[...], preferred_element_type=jnp.float32)
            + b3_ref[...])


def _encoder(R, W1r, emb, b1, W2, b2, W3, b3, tj=128):
    n, m = R.shape
    nj = m // tj
    return pl.pallas_call(
        _encoder_body,
        grid=(nj,),
        in_specs=[
            pl.BlockSpec(memory_space=pltpu.SMEM),
            pl.BlockSpec((n, tj), lambda j: (0, j)),
            pl.BlockSpec((tj, D, H1), lambda j: (j, 0, 0)),
            pl.BlockSpec((1, H1), lambda j: (0, 0)),
            pl.BlockSpec((H1, H2), lambda j: (0, 0)),
            pl.BlockSpec((1, H2), lambda j: (0, 0)),
            pl.BlockSpec((H2, D), lambda j: (0, 0)),
            pl.BlockSpec((1, D), lambda j: (0, 0)),
        ],
        out_specs=pl.BlockSpec((n, D), lambda j: (0, 0)),
        out_shape=jax.ShapeDtypeStruct((n, D), jnp.float32),
        scratch_shapes=[pltpu.VMEM((n, H1), jnp.float32)],
    )(emb, R, W1r, b1, W2, b2, W3, b3)


_EB = 2048
_NW = 32


def _sc_counts_body(aus_ref, aud_ref, avs_ref, avd_ref, cu_ref, cv_ref,
                    chunk, sbuf, dbuf):
    wid = lax.axis_index("s") * 2 + lax.axis_index("c")

    def build(src_hbm, dst_hbm, out_hbm, n_edges, n_nodes, n_passes, rows):
        nwords = rows * n_nodes
        for p in range(n_passes):
            cid = wid + _NW * p
            row0 = cid * rows

            def zero(i, _):
                chunk[pl.ds(i * 16, 16)] = jnp.zeros((16,), jnp.float32)
                return 0

            lax.fori_loop(0, nwords // 16, zero, 0)

            def blk(bi, _):
                pltpu.sync_copy(src_hbm.at[pl.ds(bi * _EB, _EB)], sbuf)
                pltpu.sync_copy(dst_hbm.at[pl.ds(bi * _EB, _EB)], dbuf)

                def grp(g, _):
                    sv = sbuf[pl.ds(g * 16, 16)]
                    dv = dbuf[pl.ds(g * 16, 16)]
                    rel = dv - row0
                    elig = (rel >= 0) & (rel < rows)
                    flat = jnp.where(elig, rel * n_nodes + sv, 0)
                    cnt, last = plsc.scan_count(flat, mask=elig)
                    plsc.addupdate_scatter(
                        chunk, [flat], cnt.astype(jnp.float32), mask=last)
                    return 0

                lax.fori_loop(0, _EB // 16, grp, 0)
                return 0

            lax.fori_loop(0, n_edges // _EB, blk, 0)

            pltpu.sync_copy(
                chunk.at[pl.ds(0, nwords)],
                out_hbm.at[pl.ds(row0 * n_nodes, nwords)])

    build(aus_ref, aud_ref, cu_ref, E_U, N_USERS, 2, 32)
    build(avs_ref, avd_ref, cv_ref, E_V, N_ITEMS, 1, 32)


def _sc_counts(au_src, au_dst, av_src, av_dst):
    mesh = plsc.VectorSubcoreMesh(core_axis_name="c", subcore_axis_name="s")
    f = pl.kernel(
        _sc_counts_body,
        out_type=(
            jax.ShapeDtypeStruct((N_USERS * N_USERS,), jnp.float32),
            jax.ShapeDtypeStruct((N_ITEMS * N_ITEMS,), jnp.float32),
        ),
        mesh=mesh,
        scratch_types=[
            pltpu.VMEM((_NW * N_USERS,), jnp.float32),
            pltpu.VMEM((_EB,), jnp.int32),
            pltpu.VMEM((_EB,), jnp.int32),
        ],
        compiler_params=pltpu.CompilerParams(needs_layout_passes=False),
    )
    return f(au_src, au_dst, av_src, av_dst)


def _gat_body(x_ref, xt_ref, C_ref, W_ref, as_ref, ad_ref, b_ref, out_ref, *,
              td):
    i = pl.program_id(0)
    x = x_ref[...]
    h = jnp.dot(x, W_ref[...], preferred_element_type=jnp.float32)
    ss = lax.dot_general(as_ref[...], h, (((1,), (1,)), ((), ())),
                         preferred_element_type=jnp.float32)
    ht = jnp.dot(xt_ref[...], W_ref[...],
                 preferred_element_type=jnp.float32)
    sd_t = jnp.sum(ht * ad_ref[...], axis=1, keepdims=True)

    n = x.shape[0]
    Z = C_ref[...]
    rid = i * td + lax.broadcasted_iota(jnp.int32, (td, n), 0)
    cid = lax.broadcasted_iota(jnp.int32, (td, n), 1)
    Z = Z + (rid == cid).astype(jnp.float32)

    e = ss + sd_t
    e = jnp.where(e >= 0, e, 0.2 * e)
    m = jnp.max(jnp.where(Z > 0, e, -1e30), axis=1, keepdims=True)
    P = Z * jnp.exp(e - m)
    den = jnp.sum(P, axis=1, keepdims=True) + 1e-16
    out = jnp.dot(P, h, preferred_element_type=jnp.float32) / den + b_ref[...]
    out_ref[...] = jnp.maximum(out, 0.0)


def _gat(x, C, W, a_s, a_d, b, td=256):
    n = x.shape[0]
    return pl.pallas_call(
        functools.partial(_gat_body, td=td),
        grid=(n // td,),
        in_specs=[
            pl.BlockSpec((n, D), lambda i: (0, 0)),
            pl.BlockSpec((td, D), lambda i: (i, 0)),
            pl.BlockSpec((td, n), lambda i: (i, 0)),
            pl.BlockSpec((D, D), lambda i: (0, 0)),
            pl.BlockSpec((1, D), lambda i: (0, 0)),
            pl.BlockSpec((1, D), lambda i: (0, 0)),
            pl.BlockSpec((1, D), lambda i: (0, 0)),
        ],
        out_specs=pl.BlockSpec((td, D), lambda i: (i, 0)),
        out_shape=jax.ShapeDtypeStruct((n, D), jnp.float32),
    )(x, x, C, W, a_s.reshape(1, D), a_d.reshape(1, D), b.reshape(1, D))


def _score_body(xu_ref, xi_ref, out_ref):
    out_ref[...] = lax.dot_general(
        xu_ref[...], xi_ref[...], (((1,), (1,)), ((), ())),
        preferred_element_type=jnp.float32)


def _score(xu, xi):
    return pl.pallas_call(
        _score_body,
        out_shape=jax.ShapeDtypeStruct((N_USERS, N_ITEMS), jnp.float32),
    )(xu, xi)


def kernel(R_U, R_V, A_U, A_V, emb, uW1, ub1, uW2, ub2, uW3, ub3, iW1, ib1,
           iW2, ib2, iW3, ib3, ug1W, ug1as, ug1ad, ug1b, ug2W, ug2as, ug2ad,
           ug2b, ig1W, ig1as, ig1ad, ig1b, ig2W, ig2as, ig2ad, ig2b):
    R_U = R_U.astype(jnp.int32)
    R_V = R_V.astype(jnp.int32)
    A_U = A_U.astype(jnp.int32)
    A_V = A_V.astype(jnp.int32)

    uW1r = uW1.reshape(N_ITEMS, D, H1)
    iW1r = iW1.reshape(N_USERS, D, H1)

    cu_flat, cv_flat = _sc_counts(A_U[0], A_U[1], A_V[0], A_V[1])
    CU = cu_flat.reshape(N_USERS, N_USERS)
    CV = cv_flat.reshape(N_ITEMS, N_ITEMS)

    xu = _encoder(R_U, uW1r, emb, ub1.reshape(1, H1), uW2,
                  ub2.reshape(1, H2), uW3, ub3.reshape(1, D))
    xi = _encoder(R_V, iW1r, emb, ib1.reshape(1, H1), iW2,
                  ib2.reshape(1, H2), iW3, ib3.reshape(1, D))

    xu = _gat(xu, CU, ug1W, ug1as, ug1ad, ug1b)
    xu = _gat(xu, CU, ug2W, ug2as, ug2ad, ug2b)
    xi = _gat(xi, CV, ig1W, ig1as, ig1ad, ig1b)
    xi = _gat(xi, CV, ig2W, ig2as, ig2ad, ig2b)

    return _score(xu, xi)


# speedup vs baseline: 68.5614x; 1.1828x over previous
import functools

import jax
import jax.numpy as jnp
from jax import lax
from jax.experimental import pallas as pl
from jax.experimental.pallas import tpu as pltpu
from jax.experimental.pallas import tpu_sc as plsc

N_USERS = 2048
N_ITEMS = 1024
D = 16
H1 = 256
H2 = 64
NR = 6
E_U = 65536
E_V = 32768


def _encoder_body(emb_ref, R_ref, W1r_ref, b1_ref, W2_ref, b2_ref, W3_ref,
                  b3_ref, out_ref, acc_ref):
    j = pl.program_id(0)
    nj = pl.num_programs(0)

    @pl.when(j == 0)
    def _():
        acc_ref[...] = jnp.zeros_like(acc_ref)

    w = W1r_ref[...]
    Rb = R_ref[...]
    acc = acc_ref[...]
    for r in range(NR):
        m_r = emb_ref[r, 0] * w[:, 0, :]
        for d in range(1, D):
            m_r = m_r + emb_ref[r, d] * w[:, d, :]
        mask = (Rb == r).astype(jnp.float32)
        acc = acc + jnp.dot(mask, m_r, preferred_element_type=jnp.float32)
    acc_ref[...] = acc

    @pl.when(j == nj - 1)
    def _():
        h = jnp.maximum(acc + b1_ref[...], 0.0)
        h = jnp.maximum(
            jnp.dot(h, W2_ref[...], preferred_element_type=jnp.float32)
            + b2_ref[...], 0.0)
        out_ref[...] = (
            jnp.dot(h, W3_ref[...], preferred_element_type=jnp.float32)
            + b3_ref[...])


def _encoder(R, W1r, emb, b1, W2, b2, W3, b3, tj=128):
    n, m = R.shape
    nj = m // tj
    return pl.pallas_call(
        _encoder_body,
        grid=(nj,),
        in_specs=[
            pl.BlockSpec(memory_space=pltpu.SMEM),
            pl.BlockSpec((n, tj), lambda j: (0, j)),
            pl.BlockSpec((tj, D, H1), lambda j: (j, 0, 0)),
            pl.BlockSpec((1, H1), lambda j: (0, 0)),
            pl.BlockSpec((H1, H2), lambda j: (0, 0)),
            pl.BlockSpec((1, H2), lambda j: (0, 0)),
            pl.BlockSpec((H2, D), lambda j: (0, 0)),
            pl.BlockSpec((1, D), lambda j: (0, 0)),
        ],
        out_specs=pl.BlockSpec((n, D), lambda j: (0, 0)),
        out_shape=jax.ShapeDtypeStruct((n, D), jnp.float32),
        scratch_shapes=[pltpu.VMEM((n, H1), jnp.float32)],
    )(emb, R, W1r, b1, W2, b2, W3, b3)


_EB = 8192
_NW = 32


def _sc_counts_body(aus_ref, aud_ref, avs_ref, avd_ref, cu_ref, cv_ref,
                    chunk, sbuf, dbuf, sems):
    wid = lax.axis_index("s") * 2 + lax.axis_index("c")

    def build(src_hbm, dst_hbm, out_hbm, n_edges, n_nodes, n_passes, rows):
        nwords = rows * n_nodes
        n_blk = n_edges // _EB

        def start(bi, slot):
            return (
                pltpu.async_copy(src_hbm.at[pl.ds(bi * _EB, _EB)],
                                 sbuf.at[pl.ds(slot * _EB, _EB)],
                                 sems.at[slot]),
                pltpu.async_copy(dst_hbm.at[pl.ds(bi * _EB, _EB)],
                                 dbuf.at[pl.ds(slot * _EB, _EB)],
                                 sems.at[slot]),
            )

        for p in range(n_passes):
            row0 = (wid + _NW * p) * rows

            def zero(i, _):
                base = i * 128
                for u in range(8):
                    chunk[pl.ds(base + u * 16, 16)] = jnp.zeros(
                        (16,), jnp.float32)
                return 0

            cps = start(0, 0)
            lax.fori_loop(0, nwords // 128, zero, 0)

            for bi in range(n_blk):
                slot = bi % 2
                for cp in cps:
                    cp.wait()
                if bi + 1 < n_blk:
                    nxt = start(bi + 1, 1 - slot)
                sbase = slot * _EB

                def grp(g, _):
                    base = sbase + g * 128
                    for u in range(8):
                        sv = sbuf[pl.ds(base + u * 16, 16)]
                        dv = dbuf[pl.ds(base + u * 16, 16)]
                        rel = dv - row0
                        elig = (rel >= 0) & (rel < rows)
                        flat = jnp.where(elig, rel * n_nodes + sv, 0)
                        cnt, last = plsc.scan_count(flat, mask=elig)
                        plsc.addupdate_scatter(
                            chunk, [flat], cnt.astype(jnp.float32), mask=last)
                    return 0

                lax.fori_loop(0, _EB // 128, grp, 0)
                if bi + 1 < n_blk:
                    cps = nxt

            pltpu.sync_copy(
                chunk.at[pl.ds(0, nwords)],
                out_hbm.at[pl.ds(row0 * n_nodes, nwords)])

    build(aus_ref, aud_ref, cu_ref, E_U, N_USERS, 2, 32)
    build(avs_ref, avd_ref, cv_ref, E_V, N_ITEMS, 1, 32)


def _sc_counts(au_src, au_dst, av_src, av_dst):
    mesh = plsc.VectorSubcoreMesh(core_axis_name="c", subcore_axis_name="s")
    f = pl.kernel(
        _sc_counts_body,
        out_type=(
            jax.ShapeDtypeStruct((N_USERS * N_USERS,), jnp.float32),
            jax.ShapeDtypeStruct((N_ITEMS * N_ITEMS,), jnp.float32),
        ),
        mesh=mesh,
        scratch_types=[
            pltpu.VMEM((_NW * N_USERS,), jnp.float32),
            pltpu.VMEM((2 * _EB,), jnp.int32),
            pltpu.VMEM((2 * _EB,), jnp.int32),
            pltpu.SemaphoreType.DMA((2,)),
        ],
        compiler_params=pltpu.CompilerParams(needs_layout_passes=False),
    )
    return f(au_src, au_dst, av_src, av_dst)


def _gat_body(x_ref, xt_ref, C_ref, W_ref, as_ref, ad_ref, b_ref, out_ref, *,
              td):
    i = pl.program_id(0)
    x = x_ref[...]
    h = jnp.dot(x, W_ref[...], preferred_element_type=jnp.float32)
    ss = lax.dot_general(as_ref[...], h, (((1,), (1,)), ((), ())),
                         preferred_element_type=jnp.float32)
    ht = jnp.dot(xt_ref[...], W_ref[...],
                 preferred_element_type=jnp.float32)
    sd_t = jnp.sum(ht * ad_ref[...], axis=1, keepdims=True)

    n = x.shape[0]
    Z = C_ref[...]
    rid = i * td + lax.broadcasted_iota(jnp.int32, (td, n), 0)
    cid = lax.broadcasted_iota(jnp.int32, (td, n), 1)
    Z = Z + (rid == cid).astype(jnp.float32)

    e = ss + sd_t
    e = jnp.where(e >= 0, e, 0.2 * e)
    m = jnp.max(jnp.where(Z > 0, e, -1e30), axis=1, keepdims=True)
    P = Z * jnp.exp(e - m)
    den = jnp.sum(P, axis=1, keepdims=True) + 1e-16
    out = jnp.dot(P, h, preferred_element_type=jnp.float32) / den + b_ref[...]
    out_ref[...] = jnp.maximum(out, 0.0)


def _gat(x, C, W, a_s, a_d, b, td=256):
    n = x.shape[0]
    return pl.pallas_call(
        functools.partial(_gat_body, td=td),
        grid=(n // td,),
        in_specs=[
            pl.BlockSpec((n, D), lambda i: (0, 0)),
            pl.BlockSpec((td, D), lambda i: (i, 0)),
            pl.BlockSpec((td, n), lambda i: (i, 0)),
            pl.BlockSpec((D, D), lambda i: (0, 0)),
            pl.BlockSpec((1, D), lambda i: (0, 0)),
            pl.BlockSpec((1, D), lambda i: (0, 0)),
            pl.BlockSpec((1, D), lambda i: (0, 0)),
        ],
        out_specs=pl.BlockSpec((td, D), lambda i: (i, 0)),
        out_shape=jax.ShapeDtypeStruct((n, D), jnp.float32),
    )(x, x, C, W, a_s.reshape(1, D), a_d.reshape(1, D), b.reshape(1, D))


def _score_body(xu_ref, xi_ref, out_ref):
    out_ref[...] = lax.dot_general(
        xu_ref[...], xi_ref[...], (((1,), (1,)), ((), ())),
        preferred_element_type=jnp.float32)


def _score(xu, xi):
    return pl.pallas_call(
        _score_body,
        out_shape=jax.ShapeDtypeStruct((N_USERS, N_ITEMS), jnp.float32),
    )(xu, xi)


def kernel(R_U, R_V, A_U, A_V, emb, uW1, ub1, uW2, ub2, uW3, ub3, iW1, ib1,
           iW2, ib2, iW3, ib3, ug1W, ug1as, ug1ad, ug1b, ug2W, ug2as, ug2ad,
           ug2b, ig1W, ig1as, ig1ad, ig1b, ig2W, ig2as, ig2ad, ig2b):
    R_U = R_U.astype(jnp.int32)
    R_V = R_V.astype(jnp.int32)
    A_U = A_U.astype(jnp.int32)
    A_V = A_V.astype(jnp.int32)

    uW1r = uW1.reshape(N_ITEMS, D, H1)
    iW1r = iW1.reshape(N_USERS, D, H1)

    cu_flat, cv_flat = _sc_counts(A_U[0], A_U[1], A_V[0], A_V[1])
    CU = cu_flat.reshape(N_USERS, N_USERS)
    CV = cv_flat.reshape(N_ITEMS, N_ITEMS)

    xu = _encoder(R_U, uW1r, emb, ub1.reshape(1, H1), uW2,
                  ub2.reshape(1, H2), uW3, ub3.reshape(1, D))
    xi = _encoder(R_V, iW1r, emb, ib1.reshape(1, H1), iW2,
                  ib2.reshape(1, H2), iW3, ib3.reshape(1, D))

    xu = _gat(xu, CU, ug1W, ug1as, ug1ad, ug1b)
    xu = _gat(xu, CU, ug2W, ug2as, ug2ad, ug2b)
    xi = _gat(xi, CV, ig1W, ig1as, ig1ad, ig1b)
    xi = _gat(xi, CV, ig2W, ig2as, ig2ad, ig2b)

    return _score(xu, xi)
